# baseline (speedup 1.0000x reference)
import functools

import jax
import jax.numpy as jnp
from jax.experimental import pallas as pl
from jax.experimental.pallas import tpu as pltpu


def _dma_copy(t, emb_ref, out_ref, sem):
    copy = pltpu.make_async_copy(emb_ref.at[pl.ds(0, t), :], out_ref, sem)
    copy.start()
    copy.wait()


def kernel(x, embedding):
    T = x.shape[1]
    F = embedding.shape[1]
    return pl.pallas_call(
        functools.partial(_dma_copy, T),
        in_specs=[pl.BlockSpec(memory_space=pl.ANY)],
        out_specs=pl.BlockSpec(memory_space=pl.ANY),
        scratch_shapes=[pltpu.SemaphoreType.DMA],
        out_shape=jax.ShapeDtypeStruct((T, F), embedding.dtype),
    )(embedding)


# speedup vs baseline: 40.8412x; 40.8412x over previous
import jax
import jax.numpy as jnp
from jax.experimental import pallas as pl


def _copy_block(emb_ref, out_ref):
    out_ref[...] = emb_ref[...]


def kernel(x, embedding):
    T = x.shape[1]
    F = embedding.shape[1]
    TILE = 1024
    return pl.pallas_call(
        _copy_block,
        grid=(T // TILE,),
        in_specs=[pl.BlockSpec((TILE, F), lambda i: (i, 0))],
        out_specs=pl.BlockSpec((TILE, F), lambda i: (i, 0)),
        out_shape=jax.ShapeDtypeStruct((T, F), embedding.dtype),
    )(embedding)


# speedup vs baseline: 44.6152x; 1.0924x over previous
import jax
import jax.numpy as jnp
from jax.experimental import pallas as pl


def _copy_block(emb_ref, out_ref):
    out_ref[...] = emb_ref[...]


def kernel(x, embedding):
    T = x.shape[1]
    F = embedding.shape[1]
    TILE = 2048
    return pl.pallas_call(
        _copy_block,
        grid=(T // TILE,),
        in_specs=[pl.BlockSpec((TILE, F), lambda i: (i, 0))],
        out_specs=pl.BlockSpec((TILE, F), lambda i: (i, 0)),
        out_shape=jax.ShapeDtypeStruct((T, F), embedding.dtype),
    )(embedding)
